# Initial kernel scaffold; baseline (speedup 1.0000x reference)
import jax
import jax.numpy as jnp
from jax.experimental import pallas as pl


def kernel(x0, x1, x2, params, edge_index0, edge_index1, edge_index2, x0_batch, x1_batch, x2_batch, assign0, assign1):
    raise NotImplementedError("write your pallas kernel here")


# speedup vs baseline: 1.7264x; 1.7264x over previous
import functools

import jax
import jax.numpy as jnp
from jax import lax
from jax.experimental import pallas as pl
from jax.experimental.pallas import tpu as pltpu
from jax.experimental.pallas import tpu_sc as plsc

N0, N1, N2, NG = 10000, 2000, 400, 16
N0P, N1P, N2P, NGP = 10240, 2048, 2048, 2048
E0P, E1P, E2P = 163840, 32768, 4096
A0E, A1E = 12288, 4096
B0E, B1E, B2E = 12288, 4096, 4096

NSC = 2
NT = 16
CHUNK = 128

_f32 = jnp.float32
_i32 = jnp.int32


def _mesh():
    return plsc.VectorSubcoreMesh(core_axis_name="c", subcore_axis_name="s",
                                  num_cores=NSC, num_subcores=NT)


def _add_offset(src_ref, dst_ref, off, n=CHUNK):
    for j in range(n // 16):
        dst_ref[pl.ds(j * 16, 16)] = src_ref[pl.ds(j * 16, 16)] + off


@functools.lru_cache(maxsize=None)
def _build_segsum(n_src_p, n_dst_p, cb, e_pad, edgesplit):
    bps = 1 if edgesplit else cb // 2
    acc_rows = bps * n_dst_p
    zr = acc_rows // NT
    assert zr % 8 == 0
    if edgesplit:
        nchunks = e_pad // NSC // NT // CHUNK
    else:
        nchunks = e_pad // NT // CHUNK

    @functools.partial(
        pl.kernel,
        out_type=jax.ShapeDtypeStruct(((2 if edgesplit else cb) * n_dst_p, 128), _f32),
        mesh=_mesh(),
        scratch_types=[
            pltpu.VMEM((CHUNK,), _i32),
            pltpu.VMEM((CHUNK,), _i32),
            pltpu.VMEM((CHUNK,), _i32),
            pltpu.VMEM((CHUNK,), _i32),
            pltpu.VMEM((CHUNK, 128), _f32),
            pltpu.VMEM((CHUNK, 128), _f32),
            pltpu.VMEM_SHARED((acc_rows, 128), _f32),
            pltpu.SemaphoreType.DMA,
        ],
    )
    def k(table, srcl, dstl, zeros128, out,
          src_v, src2_v, dst_v, dst2_v, rows_v, zrow_v, acc, sem):
        c = lax.axis_index("c")
        s = lax.axis_index("s")
        pltpu.sync_copy(zeros128, zrow_v)
        zbase = s * zr
        for t in range(zr // CHUNK):
            pltpu.sync_copy(zrow_v, acc.at[pl.ds(zbase + t * CHUNK, CHUNK)])
        if zr % CHUNK:
            pltpu.sync_copy(zrow_v.at[pl.ds(0, zr % CHUNK)],
                            acc.at[pl.ds(zbase + (zr // CHUNK) * CHUNK, zr % CHUNK)])
        plsc.subcore_barrier()
        for b in range(bps):
            if edgesplit:
                ebase = (c * (e_pad // NSC) + s * nchunks * CHUNK)
                roff = None
            else:
                ebase = s * nchunks * CHUNK
                roff = (c * bps + b) * n_src_p
            doff = b * n_dst_p

            def body(i, _, ebase=ebase, roff=roff, doff=doff):
                off = ebase + i * CHUNK
                pltpu.sync_copy(srcl.at[pl.ds(off, CHUNK)], src_v)
                pltpu.sync_copy(dstl.at[pl.ds(off, CHUNK)], dst_v)
                if roff is None:
                    gidx = src_v
                else:
                    _add_offset(src_v, src2_v, roff)
                    gidx = src2_v
                pltpu.async_copy(table.at[gidx], rows_v, sem).wait()
                if doff:
                    _add_offset(dst_v, dst2_v, doff)
                    sidx = dst2_v
                else:
                    sidx = dst_v
                pltpu.sync_copy(rows_v, acc.at[sidx], add=True)
                return _

            lax.fori_loop(0, nchunks, body, None)
        plsc.subcore_barrier()
        obase = c * acc_rows + s * zr
        pltpu.sync_copy(acc.at[pl.ds(s * zr, zr)], out.at[pl.ds(obase, zr)])

    return k


@functools.lru_cache(maxsize=None)
def _build_gather(n_src_p, n_out_p):
    rpt = n_out_p // NT
    assert rpt % 16 == 0
    nfull, tail = rpt // CHUNK, rpt % CHUNK

    @functools.partial(
        pl.kernel,
        out_type=jax.ShapeDtypeStruct((2 * n_out_p, 128), _f32),
        mesh=_mesh(),
        scratch_types=[
            pltpu.VMEM((CHUNK,), _i32),
            pltpu.VMEM((CHUNK,), _i32),
            pltpu.VMEM((CHUNK, 128), _f32),
            pltpu.SemaphoreType.DMA,
        ],
    )
    def k(table, idxl, out, idx_v, idx2_v, rows_v, sem):
        c = lax.axis_index("c")
        s = lax.axis_index("s")
        base = s * rpt
        roff = c * n_src_p

        def do_chunk(off, sz):
            pltpu.sync_copy(idxl.at[pl.ds(off, sz)], idx_v.at[pl.ds(0, sz)])
            if sz < CHUNK:
                for j in range(sz // 16, CHUNK // 16):
                    idx2_v[pl.ds(j * 16, 16)] = jnp.zeros((16,), _i32)
            _add_offset(idx_v, idx2_v, roff, n=(sz // 16) * 16)
            pltpu.async_copy(table.at[idx2_v], rows_v, sem).wait()
            pltpu.sync_copy(rows_v.at[pl.ds(0, sz)],
                            out.at[pl.ds(c * n_out_p + off, sz)])

        def body(t, _):
            do_chunk(base + t * CHUNK, CHUNK)
            return _

        lax.fori_loop(0, nfull, body, None)
        if tail:
            do_chunk(base + nfull * CHUNK, tail)

    return k


@functools.lru_cache(maxsize=None)
def _build_counts(ntot, etot):
    zr = ntot // NT
    nch = etot // NT // CHUNK
    assert zr % CHUNK == 0 and nch * NT * CHUNK == etot
    nv = zr // 16

    @functools.partial(
        pl.kernel,
        out_type=jax.ShapeDtypeStruct((ntot,), _f32),
        mesh=_mesh(),
        scratch_types=[
            pltpu.VMEM((CHUNK,), _f32),
            pltpu.VMEM((CHUNK,), _f32),
            pltpu.VMEM((CHUNK,), _i32),
            pltpu.VMEM((zr,), _f32),
            pltpu.VMEM((zr,), _f32),
            pltpu.VMEM_SHARED((ntot,), _f32),
        ],
    )
    def k(dstl, zeros1, ones1, out, z_v, one_v, dst_v, cbuf, obuf, cnt):
        c = lax.axis_index("c")
        s = lax.axis_index("s")
        pltpu.sync_copy(zeros1, z_v)
        pltpu.sync_copy(ones1, one_v)
        base = s * zr
        for t in range(zr // CHUNK):
            pltpu.sync_copy(z_v, cnt.at[pl.ds(base + t * CHUNK, CHUNK)])
        plsc.subcore_barrier()

        def body(i, _):
            off = (s * nch + i) * CHUNK
            pltpu.sync_copy(dstl.at[pl.ds(off, CHUNK)], dst_v)
            pltpu.sync_copy(one_v, cnt.at[dst_v], add=True)
            return _

        lax.fori_loop(0, nch, body, None)
        plsc.subcore_barrier()
        pltpu.sync_copy(cnt.at[pl.ds(base, zr)], cbuf)
        for j in range(nv):
            v = cbuf[pl.ds(j * 16, 16)]
            obuf[pl.ds(j * 16, 16)] = 1.0 / jnp.maximum(v, 1.0)

        @pl.when(c == 0)
        def _():
            pltpu.sync_copy(obuf, out.at[pl.ds(base, zr)])

    return k


def _tc_mm(terms, bias, np_, relu=True, bn=512):
    k = len(terms)
    has_scale = tuple(sc is not None for _, _, sc in terms)
    grid = np_ // bn

    def body(*refs):
        i = 0
        acc = None
        for j in range(k):
            m_ref = refs[i]; w_ref = refs[i + 1]; i += 2
            p = jnp.dot(m_ref[...], w_ref[...], preferred_element_type=_f32)
            if has_scale[j]:
                p = p * refs[i][...]
                i += 1
            acc = p if acc is None else acc + p
        acc = acc + refs[i][...]
        i += 1
        if relu:
            acc = jnp.maximum(acc, 0.0)
        out_ref = refs[i]
        out_ref[0] = acc[:, :128]
        out_ref[1] = acc[:, 128:]

    in_specs = []
    args = []
    for m, w, sc in terms:
        in_specs.append(pl.BlockSpec((bn, 128), lambda i: (i, 0)))
        args.append(m)
        in_specs.append(pl.BlockSpec((128, 256), lambda i: (0, 0)))
        args.append(w)
        if sc is not None:
            in_specs.append(pl.BlockSpec((bn, 1), lambda i: (i, 0)))
            args.append(sc)
    in_specs.append(pl.BlockSpec((1, 256), lambda i: (0, 0)))
    args.append(bias.reshape(1, 256))

    out = pl.pallas_call(
        body,
        grid=(grid,),
        in_specs=in_specs,
        out_specs=pl.BlockSpec((2, bn, 128), lambda i: (0, i, 0)),
        out_shape=jax.ShapeDtypeStruct((2, np_, 128), _f32),
    )(*args)
    return out.reshape(2 * np_, 128)


def _tc_head(gterms, b1, w2, b2):
    kk = len(gterms)

    def body(*refs):
        i = 0
        acc = None
        for j in range(kk):
            g = refs[i][...]; w = refs[i + 1][...]; sc = refs[i + 2][...]
            i += 3
            p = jnp.dot(g, w, preferred_element_type=_f32) * sc
            acc = p if acc is None else acc + p
        h = jnp.maximum(acc + refs[i][...], 0.0)
        out = jnp.dot(h, refs[i + 1][...], preferred_element_type=_f32) + refs[i + 2][...]
        refs[i + 3][...] = out

    args = []
    for g, w, sc in gterms:
        args += [g, w, sc]
    args += [b1.reshape(1, 256), w2, b2.reshape(1, 2)]
    return pl.pallas_call(
        body,
        out_shape=jax.ShapeDtypeStruct((128, 2), _f32),
    )(*args)


def _pad_rows(x, np_):
    return jnp.pad(x, ((0, np_ - x.shape[0]), (0, 0)))


def _colsplit(x, np_):
    n, d = x.shape
    cb = d // 128
    xp = _pad_rows(x, np_)
    return xp.reshape(np_, cb, 128).transpose(1, 0, 2).reshape(cb * np_, 128)


def _padi(idx, e_pad, fill):
    return jnp.pad(idx.astype(_i32), (0, e_pad - idx.shape[0]),
                   constant_values=fill)


def _split_w(w):
    return [w[i * 128:(i + 1) * 128] for i in range(w.shape[0] // 128)]


def kernel(x0, x1, x2, params, edge_index0, edge_index1, edge_index2,
           x0_batch, x1_batch, x2_batch, assign0, assign1):
    zeros128 = jnp.zeros((128, 128), _f32)
    zeros1 = jnp.zeros((128,), _f32)
    ones1 = jnp.ones((128,), _f32)

    X0 = _colsplit(x0, N0P)
    X1 = _colsplit(x1, N1P)
    X2 = _colsplit(x2, N2P)

    e0s = _padi(edge_index0[0], E0P, 0); e0d = _padi(edge_index0[1], E0P, N0)
    e1s = _padi(edge_index1[0], E1P, 0); e1d = _padi(edge_index1[1], E1P, N1)
    e2s = _padi(edge_index2[0], E2P, 0); e2d = _padi(edge_index2[1], E2P, N2)
    ar0 = _padi(jnp.arange(N0, dtype=_i32), A0E, 0)
    a0d = _padi(assign0, A0E, N1)
    ar1 = _padi(jnp.arange(N1, dtype=_i32), A1E, 0)
    a1d = _padi(assign1, A1E, N2)
    pb0s = _padi(jnp.arange(N0, dtype=_i32), B0E, 0)
    pb0d = _padi(x0_batch, B0E, NG)
    pb1s = _padi(jnp.arange(N1, dtype=_i32), B1E, 0)
    pb1d = _padi(x1_batch, B1E, NG)
    pb2s = _padi(jnp.arange(N2, dtype=_i32), B2E, 0)
    pb2d = _padi(x2_batch, B2E, NG)
    a0g = _padi(assign0, N0P, 0)
    a1g = _padi(assign1, N1P, 0)

    offs = [0, N0P, N0P + N1P, N0P + N1P + N2P, N0P + N1P + N2P + N1P,
            N0P + N1P + N2P + N1P + N2P]
    offs.append(offs[-1] + NGP)
    offs.append(offs[-1] + NGP)
    ntot = offs[-1] + NGP
    cat = jnp.concatenate([
        e0d, e1d + offs[1], e2d + offs[2], a0d + offs[3], a1d + offs[4],
        pb0d + offs[5], pb1d + offs[6], pb2d + offs[7]])
    inv_all = _build_counts(ntot, cat.shape[0])(cat, zeros1, ones1)
    inv_e0 = inv_all[offs[0]:offs[0] + N0P].reshape(-1, 1)
    inv_e1 = inv_all[offs[1]:offs[1] + N1P].reshape(-1, 1)
    inv_e2 = inv_all[offs[2]:offs[2] + N2P].reshape(-1, 1)
    inv_a0 = inv_all[offs[3]:offs[3] + N1P].reshape(-1, 1)
    inv_a1 = inv_all[offs[4]:offs[4] + N2P].reshape(-1, 1)
    inv_b0 = inv_all[offs[5]:offs[5] + 128].reshape(-1, 1)
    inv_b1 = inv_all[offs[6]:offs[6] + 128].reshape(-1, 1)
    inv_b2 = inv_all[offs[7]:offs[7] + 128].reshape(-1, 1)

    def blocks(h, np_, cb=2):
        return [h[i * np_:(i + 1) * np_] for i in range(cb)]

    def gcn(h, cb, np_, e_pad, srcl, dstl, inv, lp, edgesplit=False):
        seg = _build_segsum(np_, np_, cb, e_pad, edgesplit)
        s = seg(h, srcl, dstl, zeros128)
        sb = blocks(s, np_, 2 if edgesplit else cb)
        wn = _split_w(lp['Wn'])
        if edgesplit:
            wn = [wn[0], wn[0]]
        terms = [(m, w, inv) for m, w in zip(sb, wn)]
        terms += [(m, w, None) for m, w in zip(blocks(h, np_, cb), _split_w(lp['Ws']))]
        return _tc_mm(terms, lp['b'], np_)

    h0 = gcn(X0, 1, N0P, E0P, e0s, e0d, inv_e0, params['enc0_in'][0], edgesplit=True)
    h0 = gcn(h0, 2, N0P, E0P, e0s, e0d, inv_e0, params['enc0_in'][1])
    h1 = gcn(X1, 2, N1P, E1P, e1s, e1d, inv_e1, params['enc1_in'][0])
    h1 = gcn(h1, 2, N1P, E1P, e1s, e1d, inv_e1, params['enc1_in'][1])
    h2 = gcn(X2, 4, N2P, E2P, e2s, e2d, inv_e2, params['enc2_in'][0])
    h2 = gcn(h2, 2, N2P, E2P, e2s, e2d, inv_e2, params['enc2_in'][1])

    ip = params['inter']
    w0 = _split_w(ip['W0']); w1 = _split_w(ip['W1']); w2 = _split_w(ip['W2'])
    for _ in range(2):
        u01 = _build_segsum(N0P, N1P, 2, A0E, False)(h0, ar0, a0d, zeros128)
        u12 = _build_segsum(N1P, N2P, 2, A1E, False)(h1, ar1, a1d, zeros128)
        g01 = _build_gather(N1P, N0P)(h1, a0g)
        g12 = _build_gather(N2P, N1P)(h2, a1g)
        t0 = [(m, w, None) for m, w in zip(blocks(h0, N0P) + blocks(g01, N0P), w0)]
        t1 = ([(m, w, None) for m, w in zip(blocks(h1, N1P), w1[0:2])]
              + [(m, w, inv_a0) for m, w in zip(blocks(u01, N1P), w1[2:4])]
              + [(m, w, None) for m, w in zip(blocks(g12, N1P), w1[4:6])])
        t2 = ([(m, w, None) for m, w in zip(blocks(h2, N2P), w2[0:2])]
              + [(m, w, inv_a1) for m, w in zip(blocks(u12, N2P), w2[2:4])])
        h0 = _tc_mm(t0, ip['b0'], N0P)
        h1 = _tc_mm(t1, ip['b1'], N1P)
        h2 = _tc_mm(t2, ip['b2'], N2P)
        h0 = gcn(h0, 2, N0P, E0P, e0s, e0d, inv_e0, params['enc0'][0])
        h0 = gcn(h0, 2, N0P, E0P, e0s, e0d, inv_e0, params['enc0'][1])
        h1 = gcn(h1, 2, N1P, E1P, e1s, e1d, inv_e1, params['enc1'][0])
        h1 = gcn(h1, 2, N1P, E1P, e1s, e1d, inv_e1, params['enc1'][1])
        h2 = gcn(h2, 2, N2P, E2P, e2s, e2d, inv_e2, params['enc2'][0])
        h2 = gcn(h2, 2, N2P, E2P, e2s, e2d, inv_e2, params['enc2'][1])

    p0 = _build_segsum(N0P, NGP, 2, B0E, False)(h0, pb0s, pb0d, zeros128)
    p1 = _build_segsum(N1P, NGP, 2, B1E, False)(h1, pb1s, pb1d, zeros128)
    p2 = _build_segsum(N2P, NGP, 2, B2E, False)(h2, pb2s, pb2d, zeros128)

    hp = params['head']
    w1h = _split_w(hp['W1'])
    gterms = [
        (p0[0:128], w1h[0], inv_b0), (p0[NGP:NGP + 128], w1h[1], inv_b0),
        (p1[0:128], w1h[2], inv_b1), (p1[NGP:NGP + 128], w1h[3], inv_b1),
        (p2[0:128], w1h[4], inv_b2), (p2[NGP:NGP + 128], w1h[5], inv_b2),
    ]
    out = _tc_head(gterms, hp['b1'], hp['W2'], hp['b2'])
    return out[:NG]


# speedup vs baseline: 1.8047x; 1.0454x over previous
import functools

import jax
import jax.numpy as jnp
from jax import lax
from jax.experimental import pallas as pl
from jax.experimental.pallas import tpu as pltpu
from jax.experimental.pallas import tpu_sc as plsc

N0, N1, N2, NG = 10000, 2000, 400, 16
N0P, N1P, N2P, NGP = 10240, 2048, 2048, 2048
E0P, E1P, E2P = 163840, 32768, 4096
A0E, A1E = 12288, 4096
B0E, B1E, B2E = 12288, 4096, 4096

NSC = 2
NT = 16
CHUNK = 128

_f32 = jnp.float32
_i32 = jnp.int32


def _mesh():
    return plsc.VectorSubcoreMesh(core_axis_name="c", subcore_axis_name="s",
                                  num_cores=NSC, num_subcores=NT)


def _pick_nb(nchunks, cap=4):
    for nb in (cap, 2, 1):
        if nchunks % nb == 0:
            return nb
    return 1


@functools.lru_cache(maxsize=None)
def _build_segsum(n_src_p, n_dst_p, cb, e_pad, edgesplit):
    bps = 1 if edgesplit else cb // 2
    acc_rows = bps * n_dst_p
    zr = acc_rows // NT
    assert zr % 8 == 0
    total_chunks = (e_pad // NSC if edgesplit else e_pad) // CHUNK
    assert total_chunks % 8 == 0
    nchunks = 8 * max(1, total_chunks // (8 * NT))
    tiles_used = total_chunks // nchunks
    assert tiles_used * nchunks == total_chunks and tiles_used <= NT
    budget = (2097151 - acc_rows * 128 - 8192) // NT
    plan = None
    for nb, nseg in [(4, 1), (4, 2), (2, 2), (2, 4), (2, 8), (1, 1)]:
        sc_ = nchunks // nseg
        if nchunks % nseg or sc_ % nb:
            continue
        if nseg > 1 and sc_ % 8:
            continue
        if nb * 16384 + 2 * sc_ * CHUNK <= budget:
            plan = (nb, nseg, sc_)
            break
    assert plan is not None, (n_dst_p, cb, e_pad)
    nb, nseg, seg_chunks = plan
    ng = seg_chunks // nb

    @functools.partial(
        pl.kernel,
        out_type=jax.ShapeDtypeStruct(((2 if edgesplit else cb) * n_dst_p, 128), _f32),
        mesh=_mesh(),
        scratch_types=(
            [pltpu.VMEM((seg_chunks * CHUNK,), _i32),
             pltpu.VMEM((seg_chunks, CHUNK), _i32)]
            + [pltpu.VMEM((CHUNK, 128), _f32) for _ in range(nb)]
            + [pltpu.SemaphoreType.DMA for _ in range(2 * nb)]
            + [pltpu.VMEM_SHARED((acc_rows, 128), _f32)]
        ),
    )
    def k(table, srcl, dstl, zeros128, out, *scr):
        src_v, dst_v = scr[0], scr[1]
        rows = scr[2:2 + nb]
        sem_g = scr[2 + nb:2 + 2 * nb]
        sem_s = scr[2 + 2 * nb:2 + 3 * nb]
        acc = scr[2 + 3 * nb]
        c = lax.axis_index("c")
        s = lax.axis_index("s")
        pltpu.sync_copy(zeros128, rows[0])
        zbase = pl.multiple_of(s * zr, 8)
        for t in range(zr // CHUNK):
            pltpu.sync_copy(rows[0], acc.at[pl.ds(zbase + t * CHUNK, CHUNK)])
        if zr % CHUNK:
            pltpu.sync_copy(rows[0].at[pl.ds(0, zr % CHUNK)],
                            acc.at[pl.ds(zbase + (zr // CHUNK) * CHUNK, zr % CHUNK)])
        plsc.subcore_barrier()
        @pl.when(s < tiles_used)
        def _edges():
            def g_start(kc, j):
                pltpu.async_copy(table.at[src_v.at[pl.ds(kc * CHUNK, CHUNK)]],
                                 rows[j], sem_g[j])

            def g_wait(j):
                pltpu.make_async_copy(table.at[src_v.at[pl.ds(0, CHUNK)]],
                                      rows[j], sem_g[j]).wait()

            def s_start(kc, j):
                pltpu.async_copy(rows[j], acc.at[dst_v.at[kc]], sem_s[j],
                                 add=True)

            def s_wait(j):
                pltpu.make_async_copy(rows[j], acc.at[dst_v.at[0]],
                                      sem_s[j]).wait()

            def body(g, _):
                for j in range(nb):
                    g_wait(j)
                    s_start(g * nb + j, j)
                for j in range(nb):
                    s_wait(j)
                    g_start((g + 1) * nb + j, j)
                return _

            for b in range(bps):
                for seg in range(nseg):
                    if edgesplit:
                        sbase = (c * (e_pad // NSC)
                                 + (s * nchunks + seg * seg_chunks) * CHUNK)
                        dbase = (c * (e_pad // NSC // CHUNK)
                                 + s * nchunks + seg * seg_chunks)
                    else:
                        sbase = ((c * bps + b) * e_pad
                                 + (s * nchunks + seg * seg_chunks) * CHUNK)
                        dbase = (b * (e_pad // CHUNK)
                                 + s * nchunks + seg * seg_chunks)
                    sbase = pl.multiple_of(sbase, CHUNK)
                    dbase = pl.multiple_of(dbase, 8)
                    pltpu.sync_copy(srcl.at[pl.ds(sbase, seg_chunks * CHUNK)],
                                    src_v)
                    pltpu.sync_copy(dstl.at[pl.ds(dbase, seg_chunks)], dst_v)
                    for j in range(nb):
                        g_start(j, j)
                    lax.fori_loop(0, ng - 1, body, None)
                    for j in range(nb):
                        g_wait(j)
                        s_start((ng - 1) * nb + j, j)
                    for j in range(nb):
                        s_wait(j)
        plsc.subcore_barrier()
        obase = pl.multiple_of(c * acc_rows + s * zr, 8)
        pltpu.sync_copy(acc.at[pl.ds(pl.multiple_of(s * zr, 8), zr)],
                        out.at[pl.ds(obase, zr)])

    return k


@functools.lru_cache(maxsize=None)
def _build_gather(n_src_p, n_out_p):
    rpt = n_out_p // NT
    assert rpt % CHUNK == 0
    nchunks = rpt // CHUNK
    nb = min(4, nchunks)

    @functools.partial(
        pl.kernel,
        out_type=jax.ShapeDtypeStruct((2 * n_out_p, 128), _f32),
        mesh=_mesh(),
        scratch_types=(
            [pltpu.VMEM((rpt,), _i32)]
            + [pltpu.VMEM((CHUNK, 128), _f32) for _ in range(nb)]
            + [pltpu.SemaphoreType.DMA for _ in range(2 * nb)]
        ),
    )
    def k(table, idxl, out, *scr):
        idx_v = scr[0]
        rows = scr[1:1 + nb]
        sem_g = scr[1 + nb:1 + 2 * nb]
        sem_w = scr[1 + 2 * nb:1 + 3 * nb]
        c = lax.axis_index("c")
        s = lax.axis_index("s")
        base = s * rpt
        pltpu.sync_copy(idxl.at[pl.ds(c * n_out_p + base, rpt)], idx_v)
        obase = pl.multiple_of(c * n_out_p + base, 8)
        def g_start(kc, j):
            pltpu.async_copy(table.at[idx_v.at[pl.ds(kc * CHUNK, CHUNK)]],
                             rows[j], sem_g[j])

        def g_wait(j):
            pltpu.make_async_copy(table.at[idx_v.at[pl.ds(0, CHUNK)]],
                                  rows[j], sem_g[j]).wait()

        def w_wait(j):
            pltpu.make_async_copy(rows[j], out.at[pl.ds(obase, CHUNK)],
                                  sem_w[j]).wait()

        for kc in range(min(nb, nchunks)):
            g_start(kc, kc)
        pend_w = [False] * nb
        for kc in range(nchunks):
            j = kc % nb
            g_wait(j)
            pltpu.async_copy(rows[j], out.at[pl.ds(obase + kc * CHUNK, CHUNK)],
                             sem_w[j])
            pend_w[j] = True
            if kc + nb < nchunks:
                w_wait(j)
                pend_w[j] = False
                g_start(kc + nb, j)
        for j in range(nb):
            if pend_w[j]:
                w_wait(j)

    return k


@functools.lru_cache(maxsize=None)
def _build_counts(ntot, etot):
    zr = ntot // NT
    nch = etot // NT // CHUNK
    assert zr % CHUNK == 0 and nch * NT * CHUNK == etot
    nv = zr // 16

    nb = _pick_nb(nch)

    @functools.partial(
        pl.kernel,
        out_type=jax.ShapeDtypeStruct((ntot,), _f32),
        mesh=_mesh(),
        scratch_types=(
            [pltpu.VMEM((CHUNK,), _f32),
             pltpu.VMEM((CHUNK,), _f32),
             pltpu.VMEM((nch, CHUNK), _i32),
             pltpu.VMEM((zr,), _f32),
             pltpu.VMEM((zr,), _f32),
             pltpu.VMEM_SHARED((ntot,), _f32)]
            + [pltpu.SemaphoreType.DMA for _ in range(nb)]
        ),
    )
    def k(dstl, zeros1, ones1, out, z_v, one_v, dst_v, cbuf, obuf, cnt, *sems):
        c = lax.axis_index("c")
        s = lax.axis_index("s")
        pltpu.sync_copy(zeros1, z_v)
        pltpu.sync_copy(ones1, one_v)
        base = s * zr
        for t in range(zr // CHUNK):
            pltpu.sync_copy(z_v, cnt.at[pl.ds(base + t * CHUNK, CHUNK)])
        pltpu.sync_copy(dstl.at[pl.ds(pl.multiple_of(s * nch, 8), nch)], dst_v)
        plsc.subcore_barrier()

        def s_start(kc, j):
            pltpu.async_copy(one_v, cnt.at[dst_v.at[kc]], sems[j], add=True)

        def s_wait(j):
            pltpu.make_async_copy(one_v, cnt.at[dst_v.at[0]], sems[j]).wait()

        def body(g, _):
            for j in range(nb):
                s_wait(j)
                s_start((g + 1) * nb + j, j)
            return _

        for j in range(nb):
            s_start(j, j)
        lax.fori_loop(0, nch // nb - 1, body, None)
        for j in range(nb):
            s_wait(j)
        plsc.subcore_barrier()
        pltpu.sync_copy(cnt.at[pl.ds(base, zr)], cbuf)
        for j in range(nv):
            v = cbuf[pl.ds(j * 16, 16)]
            obuf[pl.ds(j * 16, 16)] = 1.0 / jnp.maximum(v, 1.0)

        @pl.when(c == 0)
        def _():
            pltpu.sync_copy(obuf, out.at[pl.ds(base, zr)])

    return k


def _tc_mm(terms, bias, np_, relu=True, bn=512):
    k = len(terms)
    has_scale = tuple(sc is not None for _, _, sc in terms)
    grid = np_ // bn

    def body(*refs):
        i = 0
        acc = None
        for j in range(k):
            m_ref = refs[i]; w_ref = refs[i + 1]; i += 2
            p = jnp.dot(m_ref[...], w_ref[...], preferred_element_type=_f32)
            if has_scale[j]:
                p = p * refs[i][...]
                i += 1
            acc = p if acc is None else acc + p
        acc = acc + refs[i][...]
        i += 1
        if relu:
            acc = jnp.maximum(acc, 0.0)
        out_ref = refs[i]
        out_ref[0] = acc[:, :128]
        out_ref[1] = acc[:, 128:]

    in_specs = []
    args = []
    for m, w, sc in terms:
        in_specs.append(pl.BlockSpec((bn, 128), lambda i: (i, 0)))
        args.append(m)
        in_specs.append(pl.BlockSpec((128, 256), lambda i: (0, 0)))
        args.append(w)
        if sc is not None:
            in_specs.append(pl.BlockSpec((bn, 1), lambda i: (i, 0)))
            args.append(sc)
    in_specs.append(pl.BlockSpec((1, 256), lambda i: (0, 0)))
    args.append(bias.reshape(1, 256))

    out = pl.pallas_call(
        body,
        grid=(grid,),
        in_specs=in_specs,
        out_specs=pl.BlockSpec((2, bn, 128), lambda i: (0, i, 0)),
        out_shape=jax.ShapeDtypeStruct((2, np_, 128), _f32),
    )(*args)
    return out.reshape(2 * np_, 128)


def _tc_head(gterms, b1, w2, b2):
    kk = len(gterms)

    def body(*refs):
        i = 0
        acc = None
        for j in range(kk):
            g = refs[i][...]; w = refs[i + 1][...]; sc = refs[i + 2][...]
            i += 3
            p = jnp.dot(g, w, preferred_element_type=_f32) * sc
            acc = p if acc is None else acc + p
        h = jnp.maximum(acc + refs[i][...], 0.0)
        out = jnp.dot(h, refs[i + 1][...], preferred_element_type=_f32) + refs[i + 2][...]
        refs[i + 3][...] = out

    args = []
    for g, w, sc in gterms:
        args += [g, w, sc]
    args += [b1.reshape(1, 256), w2, b2.reshape(1, 2)]
    return pl.pallas_call(
        body,
        out_shape=jax.ShapeDtypeStruct((128, 2), _f32),
    )(*args)


def _pad_rows(x, np_):
    return jnp.pad(x, ((0, np_ - x.shape[0]), (0, 0)))


def _colsplit(x, np_):
    n, d = x.shape
    cb = d // 128
    xp = _pad_rows(x, np_)
    return xp.reshape(np_, cb, 128).transpose(1, 0, 2).reshape(cb * np_, 128)


def _padi(idx, e_pad, fill):
    return jnp.pad(idx.astype(_i32), (0, e_pad - idx.shape[0]),
                   constant_values=fill)


def _split_w(w):
    return [w[i * 128:(i + 1) * 128] for i in range(w.shape[0] // 128)]


def kernel(x0, x1, x2, params, edge_index0, edge_index1, edge_index2,
           x0_batch, x1_batch, x2_batch, assign0, assign1):
    zeros128 = jnp.zeros((128, 128), _f32)
    zeros1 = jnp.zeros((128,), _f32)
    ones1 = jnp.ones((128,), _f32)

    X0 = _colsplit(x0, N0P)
    X1 = _colsplit(x1, N1P)
    X2 = _colsplit(x2, N2P)

    e0s = _padi(edge_index0[0], E0P, 0); e0d = _padi(edge_index0[1], E0P, N0)
    e1s = _padi(edge_index1[0], E1P, 0); e1d = _padi(edge_index1[1], E1P, N1)
    e2s = _padi(edge_index2[0], E2P, 0); e2d = _padi(edge_index2[1], E2P, N2)
    ar0 = _padi(jnp.arange(N0, dtype=_i32), A0E, 0)
    a0d = _padi(assign0, A0E, N1)
    ar1 = _padi(jnp.arange(N1, dtype=_i32), A1E, 0)
    a1d = _padi(assign1, A1E, N2)
    pb0s = _padi(jnp.arange(N0, dtype=_i32), B0E, 0)
    pb0d = _padi(x0_batch, B0E, NG)
    pb1s = _padi(jnp.arange(N1, dtype=_i32), B1E, 0)
    pb1d = _padi(x1_batch, B1E, NG)
    pb2s = _padi(jnp.arange(N2, dtype=_i32), B2E, 0)
    pb2d = _padi(x2_batch, B2E, NG)
    a0g = _padi(assign0, N0P, 0)
    a1g = _padi(assign1, N1P, 0)

    offs = [0, N0P, N0P + N1P, N0P + N1P + N2P, N0P + N1P + N2P + N1P,
            N0P + N1P + N2P + N1P + N2P]
    offs.append(offs[-1] + NGP)
    offs.append(offs[-1] + NGP)
    ntot = offs[-1] + NGP
    cat = jnp.concatenate([
        e0d, e1d + offs[1], e2d + offs[2], a0d + offs[3], a1d + offs[4],
        pb0d + offs[5], pb1d + offs[6], pb2d + offs[7]])
    etot_pad = ((cat.shape[0] + 16383) // 16384) * 16384
    cat = _padi(cat, etot_pad, ntot - 1)
    inv_all = _build_counts(ntot, etot_pad)(
        cat.reshape(-1, CHUNK), zeros1, ones1)
    inv_e0 = inv_all[offs[0]:offs[0] + N0P].reshape(-1, 1)
    inv_e1 = inv_all[offs[1]:offs[1] + N1P].reshape(-1, 1)
    inv_e2 = inv_all[offs[2]:offs[2] + N2P].reshape(-1, 1)
    inv_a0 = inv_all[offs[3]:offs[3] + N1P].reshape(-1, 1)
    inv_a1 = inv_all[offs[4]:offs[4] + N2P].reshape(-1, 1)
    inv_b0 = inv_all[offs[5]:offs[5] + 128].reshape(-1, 1)
    inv_b1 = inv_all[offs[6]:offs[6] + 128].reshape(-1, 1)
    inv_b2 = inv_all[offs[7]:offs[7] + 128].reshape(-1, 1)

    def blocks(h, np_, cb=2):
        return [h[i * np_:(i + 1) * np_] for i in range(cb)]

    def seg_idx(src, dst, cb, n_src_p, n_dst_p, edgesplit):
        if edgesplit:
            srcs, dst2 = src, dst
        else:
            srcs = jnp.concatenate([src + g * n_src_p for g in range(cb)])
            bps = cb // 2
            dst2 = (jnp.concatenate([dst + b * n_dst_p for b in range(bps)])
                    if bps > 1 else dst)
        return srcs, dst2.reshape(-1, CHUNK)

    def segsum(h, src, dst, cb, n_src_p, n_dst_p, e_pad, edgesplit=False):
        srcs, dst2 = seg_idx(src, dst, cb, n_src_p, n_dst_p, edgesplit)
        return _build_segsum(n_src_p, n_dst_p, cb, e_pad, edgesplit)(
            h, srcs, dst2, zeros128)

    def gcn(h, cb, np_, e_pad, srcl, dstl, inv, lp, edgesplit=False):
        s = segsum(h, srcl, dstl, cb, np_, np_, e_pad, edgesplit)
        sb = blocks(s, np_, 2 if edgesplit else cb)
        wn = _split_w(lp['Wn'])
        if edgesplit:
            wn = [wn[0], wn[0]]
        terms = [(m, w, inv) for m, w in zip(sb, wn)]
        terms += [(m, w, None) for m, w in zip(blocks(h, np_, cb), _split_w(lp['Ws']))]
        return _tc_mm(terms, lp['b'], np_)

    h0 = gcn(X0, 1, N0P, E0P, e0s, e0d, inv_e0, params['enc0_in'][0], edgesplit=True)
    h0 = gcn(h0, 2, N0P, E0P, e0s, e0d, inv_e0, params['enc0_in'][1])
    h1 = gcn(X1, 2, N1P, E1P, e1s, e1d, inv_e1, params['enc1_in'][0])
    h1 = gcn(h1, 2, N1P, E1P, e1s, e1d, inv_e1, params['enc1_in'][1])
    h2 = gcn(X2, 4, N2P, E2P, e2s, e2d, inv_e2, params['enc2_in'][0])
    h2 = gcn(h2, 2, N2P, E2P, e2s, e2d, inv_e2, params['enc2_in'][1])

    ip = params['inter']
    w0 = _split_w(ip['W0']); w1 = _split_w(ip['W1']); w2 = _split_w(ip['W2'])
    a0gc = jnp.concatenate([a0g, a0g + N1P])
    a1gc = jnp.concatenate([a1g, a1g + N2P])
    for _ in range(2):
        u01 = segsum(h0, ar0, a0d, 2, N0P, N1P, A0E)
        u12 = segsum(h1, ar1, a1d, 2, N1P, N2P, A1E)
        g01 = _build_gather(N1P, N0P)(h1, a0gc)
        g12 = _build_gather(N2P, N1P)(h2, a1gc)
        t0 = [(m, w, None) for m, w in zip(blocks(h0, N0P) + blocks(g01, N0P), w0)]
        t1 = ([(m, w, None) for m, w in zip(blocks(h1, N1P), w1[0:2])]
              + [(m, w, inv_a0) for m, w in zip(blocks(u01, N1P), w1[2:4])]
              + [(m, w, None) for m, w in zip(blocks(g12, N1P), w1[4:6])])
        t2 = ([(m, w, None) for m, w in zip(blocks(h2, N2P), w2[0:2])]
              + [(m, w, inv_a1) for m, w in zip(blocks(u12, N2P), w2[2:4])])
        h0 = _tc_mm(t0, ip['b0'], N0P)
        h1 = _tc_mm(t1, ip['b1'], N1P)
        h2 = _tc_mm(t2, ip['b2'], N2P)
        h0 = gcn(h0, 2, N0P, E0P, e0s, e0d, inv_e0, params['enc0'][0])
        h0 = gcn(h0, 2, N0P, E0P, e0s, e0d, inv_e0, params['enc0'][1])
        h1 = gcn(h1, 2, N1P, E1P, e1s, e1d, inv_e1, params['enc1'][0])
        h1 = gcn(h1, 2, N1P, E1P, e1s, e1d, inv_e1, params['enc1'][1])
        h2 = gcn(h2, 2, N2P, E2P, e2s, e2d, inv_e2, params['enc2'][0])
        h2 = gcn(h2, 2, N2P, E2P, e2s, e2d, inv_e2, params['enc2'][1])

    p0 = segsum(h0, pb0s, pb0d, 2, N0P, NGP, B0E)
    p1 = segsum(h1, pb1s, pb1d, 2, N1P, NGP, B1E)
    p2 = segsum(h2, pb2s, pb2d, 2, N2P, NGP, B2E)

    hp = params['head']
    w1h = _split_w(hp['W1'])
    gterms = [
        (p0[0:128], w1h[0], inv_b0), (p0[NGP:NGP + 128], w1h[1], inv_b0),
        (p1[0:128], w1h[2], inv_b1), (p1[NGP:NGP + 128], w1h[3], inv_b1),
        (p2[0:128], w1h[4], inv_b2), (p2[NGP:NGP + 128], w1h[5], inv_b2),
    ]
    out = _tc_head(gterms, hp['b1'], hp['W2'], hp['b2'])
    return out[:NG]


# speedup vs baseline: 2.3851x; 1.3216x over previous
import functools

import jax
import jax.numpy as jnp
from jax import lax
from jax.experimental import pallas as pl
from jax.experimental.pallas import tpu as pltpu
from jax.experimental.pallas import tpu_sc as plsc

N0, N1, N2, NG = 10000, 2000, 400, 16
N0P, N1P, N2P, NGP = 10240, 2048, 2048, 2048
E0P, E1P, E2P = 163840, 32768, 4096
A0E, A1E = 10240, 2048
B0E, B1E, B2E = 10240, 2048, 2048

NSC = 2
NT = 16
CHUNK = 128

_f32 = jnp.float32
_i32 = jnp.int32


def _mesh():
    return plsc.VectorSubcoreMesh(core_axis_name="c", subcore_axis_name="s",
                                  num_cores=NSC, num_subcores=NT)


def _pick_nb(nchunks, cap=4):
    for nb in (cap, 2, 1):
        if nchunks % nb == 0:
            return nb
    return 1


@functools.lru_cache(maxsize=None)
def _build_segsum(n_src_p, n_dst_p, cb, e_pad, edgesplit, linear_src=False):
    bps = 1 if edgesplit else cb // 2
    acc_rows = bps * n_dst_p
    zr = acc_rows // NT
    assert zr % 8 == 0
    total_chunks = (e_pad // NSC if edgesplit else e_pad) // CHUNK
    assert total_chunks % 8 == 0
    nchunks = 8 * max(1, total_chunks // (8 * NT))
    tiles_used = total_chunks // nchunks
    assert tiles_used * nchunks == total_chunks and tiles_used <= NT
    budget = (2097151 - acc_rows * 128 - 8192) // NT
    plan = None
    for nb, nseg in [(4, 1), (4, 2), (2, 2), (2, 4), (2, 8), (1, 1)]:
        sc_ = nchunks // nseg
        if nchunks % nseg or sc_ % nb:
            continue
        if nseg > 1 and sc_ % 8:
            continue
        if nb * 16384 + 2 * sc_ * CHUNK <= budget:
            plan = (nb, nseg, sc_)
            break
    assert plan is not None, (n_dst_p, cb, e_pad)
    nb, nseg, seg_chunks = plan
    ng = seg_chunks // nb
    if linear_src:
        assert e_pad == n_src_p and not edgesplit

    @functools.partial(
        pl.kernel,
        out_type=jax.ShapeDtypeStruct(((2 if edgesplit else cb) * n_dst_p, 128), _f32),
        mesh=_mesh(),
        scratch_types=(
            [pltpu.VMEM((seg_chunks * CHUNK,), _i32),
             pltpu.VMEM((seg_chunks, CHUNK), _i32)]
            + [pltpu.VMEM((CHUNK, 128), _f32) for _ in range(nb)]
            + [pltpu.SemaphoreType.DMA for _ in range(2 * nb)]
            + [pltpu.VMEM_SHARED((acc_rows, 128), _f32)]
        ),
    )
    def k(table, srcl, dstl, zeros128, out, *scr):
        src_v, dst_v = scr[0], scr[1]
        rows = scr[2:2 + nb]
        sem_g = scr[2 + nb:2 + 2 * nb]
        sem_s = scr[2 + 2 * nb:2 + 3 * nb]
        acc = scr[2 + 3 * nb]
        c = lax.axis_index("c")
        s = lax.axis_index("s")
        pltpu.sync_copy(zeros128, rows[0])
        zbase = pl.multiple_of(s * zr, 8)
        for t in range(zr // CHUNK):
            pltpu.sync_copy(rows[0], acc.at[pl.ds(zbase + t * CHUNK, CHUNK)])
        if zr % CHUNK:
            pltpu.sync_copy(rows[0].at[pl.ds(0, zr % CHUNK)],
                            acc.at[pl.ds(zbase + (zr // CHUNK) * CHUNK, zr % CHUNK)])
        plsc.subcore_barrier()
        @pl.when(s < tiles_used)
        def _edges():
            lin_base = [0]

            def g_start(kc, j):
                if linear_src:
                    off = pl.multiple_of(lin_base[0] + kc * CHUNK, CHUNK)
                    pltpu.async_copy(table.at[pl.ds(off, CHUNK)],
                                     rows[j], sem_g[j])
                else:
                    pltpu.async_copy(table.at[src_v.at[pl.ds(kc * CHUNK, CHUNK)]],
                                     rows[j], sem_g[j])

            def g_wait(j):
                if linear_src:
                    pltpu.make_async_copy(table.at[pl.ds(0, CHUNK)],
                                          rows[j], sem_g[j]).wait()
                else:
                    pltpu.make_async_copy(table.at[src_v.at[pl.ds(0, CHUNK)]],
                                          rows[j], sem_g[j]).wait()

            def s_start(kc, j):
                pltpu.async_copy(rows[j], acc.at[dst_v.at[kc]], sem_s[j],
                                 add=True)

            def s_wait(j):
                pltpu.make_async_copy(rows[j], acc.at[dst_v.at[0]],
                                      sem_s[j]).wait()

            def body(g, _):
                for j in range(nb):
                    g_wait(j)
                    s_start(g * nb + j, j)
                for j in range(nb):
                    s_wait(j)
                    g_start((g + 1) * nb + j, j)
                return _

            for b in range(bps):
                for seg in range(nseg):
                    if edgesplit:
                        sbase = (c * (e_pad // NSC)
                                 + (s * nchunks + seg * seg_chunks) * CHUNK)
                        dbase = (c * (e_pad // NSC // CHUNK)
                                 + s * nchunks + seg * seg_chunks)
                    else:
                        sbase = ((c * bps + b) * e_pad
                                 + (s * nchunks + seg * seg_chunks) * CHUNK)
                        dbase = (b * (e_pad // CHUNK)
                                 + s * nchunks + seg * seg_chunks)
                    sbase = pl.multiple_of(sbase, CHUNK)
                    dbase = pl.multiple_of(dbase, 8)
                    if linear_src:
                        lin_base[0] = sbase
                    else:
                        pltpu.sync_copy(srcl.at[pl.ds(sbase, seg_chunks * CHUNK)],
                                        src_v)
                    pltpu.sync_copy(dstl.at[pl.ds(dbase, seg_chunks)], dst_v)
                    for j in range(nb):
                        g_start(j, j)
                    lax.fori_loop(0, ng - 1, body, None)
                    for j in range(nb):
                        g_wait(j)
                        s_start((ng - 1) * nb + j, j)
                    for j in range(nb):
                        s_wait(j)
        plsc.subcore_barrier()
        obase = pl.multiple_of(c * acc_rows + s * zr, 8)
        pltpu.sync_copy(acc.at[pl.ds(pl.multiple_of(s * zr, 8), zr)],
                        out.at[pl.ds(obase, zr)])

    return k


@functools.lru_cache(maxsize=None)
def _build_gather(n_src_p, n_out_p):
    rpt = n_out_p // NT
    assert rpt % CHUNK == 0
    nchunks = rpt // CHUNK
    nb = min(4, nchunks)

    @functools.partial(
        pl.kernel,
        out_type=jax.ShapeDtypeStruct((2 * n_out_p, 128), _f32),
        mesh=_mesh(),
        scratch_types=(
            [pltpu.VMEM((rpt,), _i32)]
            + [pltpu.VMEM((CHUNK, 128), _f32) for _ in range(nb)]
            + [pltpu.SemaphoreType.DMA for _ in range(2 * nb)]
        ),
    )
    def k(table, idxl, out, *scr):
        idx_v = scr[0]
        rows = scr[1:1 + nb]
        sem_g = scr[1 + nb:1 + 2 * nb]
        sem_w = scr[1 + 2 * nb:1 + 3 * nb]
        c = lax.axis_index("c")
        s = lax.axis_index("s")
        base = s * rpt
        pltpu.sync_copy(idxl.at[pl.ds(c * n_out_p + base, rpt)], idx_v)
        obase = pl.multiple_of(c * n_out_p + base, 8)
        def g_start(kc, j):
            pltpu.async_copy(table.at[idx_v.at[pl.ds(kc * CHUNK, CHUNK)]],
                             rows[j], sem_g[j])

        def g_wait(j):
            pltpu.make_async_copy(table.at[idx_v.at[pl.ds(0, CHUNK)]],
                                  rows[j], sem_g[j]).wait()

        def w_wait(j):
            pltpu.make_async_copy(rows[j], out.at[pl.ds(obase, CHUNK)],
                                  sem_w[j]).wait()

        for kc in range(min(nb, nchunks)):
            g_start(kc, kc)
        pend_w = [False] * nb
        for kc in range(nchunks):
            j = kc % nb
            g_wait(j)
            pltpu.async_copy(rows[j], out.at[pl.ds(obase + kc * CHUNK, CHUNK)],
                             sem_w[j])
            pend_w[j] = True
            if kc + nb < nchunks:
                w_wait(j)
                pend_w[j] = False
                g_start(kc + nb, j)
        for j in range(nb):
            if pend_w[j]:
                w_wait(j)

    return k


@functools.lru_cache(maxsize=None)
def _build_counts(ntot, etot):
    zr = ntot // NT
    nch = etot // NT // CHUNK
    assert zr % CHUNK == 0 and nch * NT * CHUNK == etot
    nv = zr // 16

    nb = _pick_nb(nch)

    @functools.partial(
        pl.kernel,
        out_type=jax.ShapeDtypeStruct((ntot,), _f32),
        mesh=_mesh(),
        scratch_types=(
            [pltpu.VMEM((CHUNK,), _f32),
             pltpu.VMEM((CHUNK,), _f32),
             pltpu.VMEM((nch, CHUNK), _i32),
             pltpu.VMEM((zr,), _f32),
             pltpu.VMEM((zr,), _f32),
             pltpu.VMEM_SHARED((ntot,), _f32)]
            + [pltpu.SemaphoreType.DMA for _ in range(nb)]
        ),
    )
    def k(dstl, zeros1, ones1, out, z_v, one_v, dst_v, cbuf, obuf, cnt, *sems):
        c = lax.axis_index("c")
        s = lax.axis_index("s")
        pltpu.sync_copy(zeros1, z_v)
        pltpu.sync_copy(ones1, one_v)
        base = s * zr
        for t in range(zr // CHUNK):
            pltpu.sync_copy(z_v, cnt.at[pl.ds(base + t * CHUNK, CHUNK)])
        pltpu.sync_copy(dstl.at[pl.ds(pl.multiple_of(s * nch, 8), nch)], dst_v)
        plsc.subcore_barrier()

        def s_start(kc, j):
            pltpu.async_copy(one_v, cnt.at[dst_v.at[kc]], sems[j], add=True)

        def s_wait(j):
            pltpu.make_async_copy(one_v, cnt.at[dst_v.at[0]], sems[j]).wait()

        def body(g, _):
            for j in range(nb):
                s_wait(j)
                s_start((g + 1) * nb + j, j)
            return _

        for j in range(nb):
            s_start(j, j)
        lax.fori_loop(0, nch // nb - 1, body, None)
        for j in range(nb):
            s_wait(j)
        plsc.subcore_barrier()
        pltpu.sync_copy(cnt.at[pl.ds(base, zr)], cbuf)
        for j in range(nv):
            v = cbuf[pl.ds(j * 16, 16)]
            obuf[pl.ds(j * 16, 16)] = 1.0 / jnp.maximum(v, 1.0)

        @pl.when(c == 0)
        def _():
            pltpu.sync_copy(obuf, out.at[pl.ds(base, zr)])

    return k


def _tc_mm(terms, bias, np_, relu=True, bn=512):
    k = len(terms)
    has_scale = tuple(sc is not None for _, _, sc in terms)
    grid = np_ // bn

    def body(*refs):
        i = 0
        acc = None
        for j in range(k):
            m_ref = refs[i]; w_ref = refs[i + 1]; i += 2
            p = jnp.dot(m_ref[...], w_ref[...], preferred_element_type=_f32)
            if has_scale[j]:
                p = p * refs[i][...]
                i += 1
            acc = p if acc is None else acc + p
        acc = acc + refs[i][...]
        i += 1
        if relu:
            acc = jnp.maximum(acc, 0.0)
        out_ref = refs[i]
        out_ref[0] = acc[:, :128]
        out_ref[1] = acc[:, 128:]

    in_specs = []
    args = []
    for m, w, sc in terms:
        in_specs.append(pl.BlockSpec((bn, 128), lambda i: (i, 0)))
        args.append(m)
        in_specs.append(pl.BlockSpec((128, 256), lambda i: (0, 0)))
        args.append(w)
        if sc is not None:
            in_specs.append(pl.BlockSpec((bn, 1), lambda i: (i, 0)))
            args.append(sc)
    in_specs.append(pl.BlockSpec((1, 256), lambda i: (0, 0)))
    args.append(bias.reshape(1, 256))

    out = pl.pallas_call(
        body,
        grid=(grid,),
        in_specs=in_specs,
        out_specs=pl.BlockSpec((2, bn, 128), lambda i: (0, i, 0)),
        out_shape=jax.ShapeDtypeStruct((2, np_, 128), _f32),
    )(*args)
    return out.reshape(2 * np_, 128)


def _tc_head(gterms, b1, w2, b2):
    kk = len(gterms)

    def body(*refs):
        i = 0
        acc = None
        for j in range(kk):
            g = refs[i][...]; w = refs[i + 1][...]; sc = refs[i + 2][...]
            i += 3
            p = jnp.dot(g, w, preferred_element_type=_f32) * sc
            acc = p if acc is None else acc + p
        h = jnp.maximum(acc + refs[i][...], 0.0)
        out = jnp.dot(h, refs[i + 1][...], preferred_element_type=_f32) + refs[i + 2][...]
        refs[i + 3][...] = out

    args = []
    for g, w, sc in gterms:
        args += [g, w, sc]
    args += [b1.reshape(1, 256), w2, b2.reshape(1, 2)]
    return pl.pallas_call(
        body,
        out_shape=jax.ShapeDtypeStruct((128, 2), _f32),
    )(*args)


def _pad_rows(x, np_):
    return jnp.pad(x, ((0, np_ - x.shape[0]), (0, 0)))


def _colsplit(x, np_):
    n, d = x.shape
    cb = d // 128
    xp = _pad_rows(x, np_)
    return xp.reshape(np_, cb, 128).transpose(1, 0, 2).reshape(cb * np_, 128)


def _padi(idx, e_pad, fill):
    return jnp.pad(idx.astype(_i32), (0, e_pad - idx.shape[0]),
                   constant_values=fill)


def _split_w(w):
    return [w[i * 128:(i + 1) * 128] for i in range(w.shape[0] // 128)]


def kernel(x0, x1, x2, params, edge_index0, edge_index1, edge_index2,
           x0_batch, x1_batch, x2_batch, assign0, assign1):
    zeros128 = jnp.zeros((128, 128), _f32)
    zeros1 = jnp.zeros((128,), _f32)
    ones1 = jnp.ones((128,), _f32)

    X0 = _colsplit(x0, N0P)
    X1 = _colsplit(x1, N1P)
    X2 = _colsplit(x2, N2P)

    e0s = _padi(edge_index0[0], E0P, 0); e0d = _padi(edge_index0[1], E0P, N0)
    e1s = _padi(edge_index1[0], E1P, 0); e1d = _padi(edge_index1[1], E1P, N1)
    e2s = _padi(edge_index2[0], E2P, 0); e2d = _padi(edge_index2[1], E2P, N2)
    ar0 = _padi(jnp.arange(N0, dtype=_i32), A0E, 0)
    a0d = _padi(assign0, A0E, N1)
    ar1 = _padi(jnp.arange(N1, dtype=_i32), A1E, 0)
    a1d = _padi(assign1, A1E, N2)
    pb0s = _padi(jnp.arange(N0, dtype=_i32), B0E, 0)
    pb0d = _padi(x0_batch, B0E, NG)
    pb1s = _padi(jnp.arange(N1, dtype=_i32), B1E, 0)
    pb1d = _padi(x1_batch, B1E, NG)
    pb2s = _padi(jnp.arange(N2, dtype=_i32), B2E, 0)
    pb2d = _padi(x2_batch, B2E, NG)
    a0g = _padi(assign0, N0P, 0)
    a1g = _padi(assign1, N1P, 0)

    offs = [0, N0P, N0P + N1P, N0P + N1P + N2P, N0P + N1P + N2P + N1P,
            N0P + N1P + N2P + N1P + N2P]
    offs.append(offs[-1] + NGP)
    offs.append(offs[-1] + NGP)
    ntot = offs[-1] + NGP
    cat = jnp.concatenate([
        e0d, e1d + offs[1], e2d + offs[2], a0d + offs[3], a1d + offs[4],
        pb0d + offs[5], pb1d + offs[6], pb2d + offs[7]])
    etot_pad = ((cat.shape[0] + 16383) // 16384) * 16384
    cat = _padi(cat, etot_pad, ntot - 1)
    inv_all = _build_counts(ntot, etot_pad)(
        cat.reshape(-1, CHUNK), zeros1, ones1)
    inv_e0 = inv_all[offs[0]:offs[0] + N0P].reshape(-1, 1)
    inv_e1 = inv_all[offs[1]:offs[1] + N1P].reshape(-1, 1)
    inv_e2 = inv_all[offs[2]:offs[2] + N2P].reshape(-1, 1)
    inv_a0 = inv_all[offs[3]:offs[3] + N1P].reshape(-1, 1)
    inv_a1 = inv_all[offs[4]:offs[4] + N2P].reshape(-1, 1)
    inv_b0 = inv_all[offs[5]:offs[5] + 128].reshape(-1, 1)
    inv_b1 = inv_all[offs[6]:offs[6] + 128].reshape(-1, 1)
    inv_b2 = inv_all[offs[7]:offs[7] + 128].reshape(-1, 1)

    def blocks(h, np_, cb=2):
        return [h[i * np_:(i + 1) * np_] for i in range(cb)]

    def seg_idx(src, dst, cb, n_src_p, n_dst_p, edgesplit):
        if edgesplit:
            srcs, dst2 = src, dst
        else:
            srcs = jnp.concatenate([src + g * n_src_p for g in range(cb)])
            bps = cb // 2
            dst2 = (jnp.concatenate([dst + b * n_dst_p for b in range(bps)])
                    if bps > 1 else dst)
        return srcs, dst2.reshape(-1, CHUNK)

    def segsum(h, src, dst, cb, n_src_p, n_dst_p, e_pad, edgesplit=False,
               linear_src=False):
        srcs, dst2 = seg_idx(src, dst, cb, n_src_p, n_dst_p, edgesplit)
        if linear_src:
            srcs = srcs[:CHUNK]
        return _build_segsum(n_src_p, n_dst_p, cb, e_pad, edgesplit,
                             linear_src)(h, srcs, dst2, zeros128)

    def gcn(h, cb, np_, e_pad, srcl, dstl, inv, lp, edgesplit=False):
        s = segsum(h, srcl, dstl, cb, np_, np_, e_pad, edgesplit)
        sb = blocks(s, np_, 2 if edgesplit else cb)
        wn = _split_w(lp['Wn'])
        if edgesplit:
            wn = [wn[0], wn[0]]
        terms = [(m, w, inv) for m, w in zip(sb, wn)]
        terms += [(m, w, None) for m, w in zip(blocks(h, np_, cb), _split_w(lp['Ws']))]
        return _tc_mm(terms, lp['b'], np_)

    h0 = gcn(X0, 1, N0P, E0P, e0s, e0d, inv_e0, params['enc0_in'][0], edgesplit=True)
    h0 = gcn(h0, 2, N0P, E0P, e0s, e0d, inv_e0, params['enc0_in'][1])
    h1 = gcn(X1, 2, N1P, E1P, e1s, e1d, inv_e1, params['enc1_in'][0])
    h1 = gcn(h1, 2, N1P, E1P, e1s, e1d, inv_e1, params['enc1_in'][1])
    h2 = gcn(X2, 4, N2P, E2P, e2s, e2d, inv_e2, params['enc2_in'][0])
    h2 = gcn(h2, 2, N2P, E2P, e2s, e2d, inv_e2, params['enc2_in'][1])

    ip = params['inter']
    w0 = _split_w(ip['W0']); w1 = _split_w(ip['W1']); w2 = _split_w(ip['W2'])
    a0gc = jnp.concatenate([a0g, a0g + N1P])
    a1gc = jnp.concatenate([a1g, a1g + N2P])
    for _ in range(2):
        u01 = segsum(h0, ar0, a0d, 2, N0P, N1P, A0E, linear_src=True)
        u12 = segsum(h1, ar1, a1d, 2, N1P, N2P, A1E, linear_src=True)
        g01 = _build_gather(N1P, N0P)(h1, a0gc)
        g12 = _build_gather(N2P, N1P)(h2, a1gc)
        t0 = [(m, w, None) for m, w in zip(blocks(h0, N0P) + blocks(g01, N0P), w0)]
        t1 = ([(m, w, None) for m, w in zip(blocks(h1, N1P), w1[0:2])]
              + [(m, w, inv_a0) for m, w in zip(blocks(u01, N1P), w1[2:4])]
              + [(m, w, None) for m, w in zip(blocks(g12, N1P), w1[4:6])])
        t2 = ([(m, w, None) for m, w in zip(blocks(h2, N2P), w2[0:2])]
              + [(m, w, inv_a1) for m, w in zip(blocks(u12, N2P), w2[2:4])])
        h0 = _tc_mm(t0, ip['b0'], N0P)
        h1 = _tc_mm(t1, ip['b1'], N1P)
        h2 = _tc_mm(t2, ip['b2'], N2P)
        h0 = gcn(h0, 2, N0P, E0P, e0s, e0d, inv_e0, params['enc0'][0])
        h0 = gcn(h0, 2, N0P, E0P, e0s, e0d, inv_e0, params['enc0'][1])
        h1 = gcn(h1, 2, N1P, E1P, e1s, e1d, inv_e1, params['enc1'][0])
        h1 = gcn(h1, 2, N1P, E1P, e1s, e1d, inv_e1, params['enc1'][1])
        h2 = gcn(h2, 2, N2P, E2P, e2s, e2d, inv_e2, params['enc2'][0])
        h2 = gcn(h2, 2, N2P, E2P, e2s, e2d, inv_e2, params['enc2'][1])

    p0 = segsum(h0, pb0s, pb0d, 2, N0P, NGP, B0E, linear_src=True)
    p1 = segsum(h1, pb1s, pb1d, 2, N1P, NGP, B1E, linear_src=True)
    p2 = segsum(h2, pb2s, pb2d, 2, N2P, NGP, B2E, linear_src=True)

    hp = params['head']
    w1h = _split_w(hp['W1'])
    gterms = [
        (p0[0:128], w1h[0], inv_b0), (p0[NGP:NGP + 128], w1h[1], inv_b0),
        (p1[0:128], w1h[2], inv_b1), (p1[NGP:NGP + 128], w1h[3], inv_b1),
        (p2[0:128], w1h[4], inv_b2), (p2[NGP:NGP + 128], w1h[5], inv_b2),
    ]
    out = _tc_head(gterms, hp['b1'], hp['W2'], hp['b2'])
    return out[:NG]


# speedup vs baseline: 2.3866x; 1.0007x over previous
import functools

import jax
import jax.numpy as jnp
from jax import lax
from jax.experimental import pallas as pl
from jax.experimental.pallas import tpu as pltpu
from jax.experimental.pallas import tpu_sc as plsc

N0, N1, N2, NG = 10000, 2000, 400, 16
N0P, N1P, N2P, NGP = 10240, 2048, 2048, 2048
E0P, E1P, E2P = 163840, 32768, 4096
A0E, A1E = 10240, 2048
B0E, B1E, B2E = 10240, 2048, 2048

NSC = 2
NT = 16
CHUNK = 128

_f32 = jnp.float32
_i32 = jnp.int32


def _mesh():
    return plsc.VectorSubcoreMesh(core_axis_name="c", subcore_axis_name="s",
                                  num_cores=NSC, num_subcores=NT)


def _pick_nb(nchunks, cap=4):
    for nb in (cap, 2, 1):
        if nchunks % nb == 0:
            return nb
    return 1


@functools.lru_cache(maxsize=None)
def _build_segsum(n_src_p, n_dst_p, cb, e_pad, edgesplit, linear_src=False):
    bps = 1 if edgesplit else cb // 2
    acc_rows = bps * n_dst_p
    zr = acc_rows // NT
    assert zr % 8 == 0
    total_chunks = (e_pad // NSC if edgesplit else e_pad) // CHUNK
    assert total_chunks % 8 == 0
    nchunks = 8 * max(1, total_chunks // (8 * NT))
    tiles_used = total_chunks // nchunks
    assert tiles_used * nchunks == total_chunks and tiles_used <= NT
    budget = (2097151 - acc_rows * 128 - 8192) // NT
    plan = None
    for nb, nseg in [(4, 1), (4, 2), (2, 2), (2, 4), (2, 8), (1, 1)]:
        sc_ = nchunks // nseg
        if nchunks % nseg or sc_ % nb:
            continue
        if nseg > 1 and sc_ % 8:
            continue
        if nb * 16384 + 2 * sc_ * CHUNK <= budget:
            plan = (nb, nseg, sc_)
            break
    assert plan is not None, (n_dst_p, cb, e_pad)
    nb, nseg, seg_chunks = plan
    ng = seg_chunks // nb
    if linear_src:
        assert e_pad == n_src_p and not edgesplit

    @functools.partial(
        pl.kernel,
        out_type=jax.ShapeDtypeStruct(((2 if edgesplit else cb) * n_dst_p, 128), _f32),
        mesh=_mesh(),
        scratch_types=(
            [pltpu.VMEM((seg_chunks * CHUNK,), _i32),
             pltpu.VMEM((seg_chunks, CHUNK), _i32)]
            + [pltpu.VMEM((CHUNK, 128), _f32) for _ in range(nb)]
            + [pltpu.SemaphoreType.DMA for _ in range(2 * nb)]
            + [pltpu.VMEM_SHARED((acc_rows, 128), _f32)]
        ),
    )
    def k(table, srcl, dstl, zeros128, out, *scr):
        src_v, dst_v = scr[0], scr[1]
        rows = scr[2:2 + nb]
        sem_g = scr[2 + nb:2 + 2 * nb]
        sem_s = scr[2 + 2 * nb:2 + 3 * nb]
        acc = scr[2 + 3 * nb]
        c = lax.axis_index("c")
        s = lax.axis_index("s")
        pltpu.sync_copy(zeros128, rows[0])
        zbase = pl.multiple_of(s * zr, 8)
        for t in range(zr // CHUNK):
            pltpu.sync_copy(rows[0], acc.at[pl.ds(zbase + t * CHUNK, CHUNK)])
        if zr % CHUNK:
            pltpu.sync_copy(rows[0].at[pl.ds(0, zr % CHUNK)],
                            acc.at[pl.ds(zbase + (zr // CHUNK) * CHUNK, zr % CHUNK)])
        plsc.subcore_barrier()
        @pl.when(s < tiles_used)
        def _edges():
            lin_base = [0]

            def g_start(kc, j):
                if linear_src:
                    off = pl.multiple_of(lin_base[0] + kc * CHUNK, CHUNK)
                    pltpu.async_copy(table.at[pl.ds(off, CHUNK)],
                                     rows[j], sem_g[j])
                else:
                    pltpu.async_copy(table.at[src_v.at[pl.ds(kc * CHUNK, CHUNK)]],
                                     rows[j], sem_g[j])

            def g_wait(j):
                if linear_src:
                    pltpu.make_async_copy(table.at[pl.ds(0, CHUNK)],
                                          rows[j], sem_g[j]).wait()
                else:
                    pltpu.make_async_copy(table.at[src_v.at[pl.ds(0, CHUNK)]],
                                          rows[j], sem_g[j]).wait()

            def s_start(kc, j):
                pltpu.async_copy(rows[j], acc.at[dst_v.at[kc]], sem_s[j],
                                 add=True)

            def s_wait(j):
                pltpu.make_async_copy(rows[j], acc.at[dst_v.at[0]],
                                      sem_s[j]).wait()

            def body(g, _):
                for j in range(nb):
                    g_wait(j)
                    s_start(g * nb + j, j)
                for j in range(nb):
                    s_wait(j)
                    g_start((g + 1) * nb + j, j)
                return _

            for b in range(bps):
                for seg in range(nseg):
                    if edgesplit:
                        sbase = (c * (e_pad // NSC)
                                 + (s * nchunks + seg * seg_chunks) * CHUNK)
                        dbase = (c * (e_pad // NSC // CHUNK)
                                 + s * nchunks + seg * seg_chunks)
                    else:
                        sbase = ((c * bps + b) * e_pad
                                 + (s * nchunks + seg * seg_chunks) * CHUNK)
                        dbase = (b * (e_pad // CHUNK)
                                 + s * nchunks + seg * seg_chunks)
                    sbase = pl.multiple_of(sbase, CHUNK)
                    dbase = pl.multiple_of(dbase, 8)
                    if linear_src:
                        lin_base[0] = sbase
                    else:
                        pltpu.sync_copy(srcl.at[pl.ds(sbase, seg_chunks * CHUNK)],
                                        src_v)
                    pltpu.sync_copy(dstl.at[pl.ds(dbase, seg_chunks)], dst_v)
                    for j in range(nb):
                        g_start(j, j)
                    lax.fori_loop(0, ng - 1, body, None)
                    for j in range(nb):
                        g_wait(j)
                        s_start((ng - 1) * nb + j, j)
                    for j in range(nb):
                        s_wait(j)
        plsc.subcore_barrier()
        obase = pl.multiple_of(c * acc_rows + s * zr, 8)
        pltpu.sync_copy(acc.at[pl.ds(pl.multiple_of(s * zr, 8), zr)],
                        out.at[pl.ds(obase, zr)])

    return k


@functools.lru_cache(maxsize=None)
def _build_gather(n_src_p, n_out_p):
    rpt = n_out_p // NT
    assert rpt % CHUNK == 0
    nchunks = rpt // CHUNK
    nb = min(4, nchunks)

    @functools.partial(
        pl.kernel,
        out_type=jax.ShapeDtypeStruct((2 * n_out_p, 128), _f32),
        mesh=_mesh(),
        scratch_types=(
            [pltpu.VMEM((rpt,), _i32)]
            + [pltpu.VMEM((CHUNK, 128), _f32) for _ in range(nb)]
            + [pltpu.SemaphoreType.DMA for _ in range(2 * nb)]
        ),
    )
    def k(table, idxl, out, *scr):
        idx_v = scr[0]
        rows = scr[1:1 + nb]
        sem_g = scr[1 + nb:1 + 2 * nb]
        sem_w = scr[1 + 2 * nb:1 + 3 * nb]
        c = lax.axis_index("c")
        s = lax.axis_index("s")
        base = s * rpt
        pltpu.sync_copy(idxl.at[pl.ds(c * n_out_p + base, rpt)], idx_v)
        obase = pl.multiple_of(c * n_out_p + base, 8)
        def g_start(kc, j):
            pltpu.async_copy(table.at[idx_v.at[pl.ds(kc * CHUNK, CHUNK)]],
                             rows[j], sem_g[j])

        def g_wait(j):
            pltpu.make_async_copy(table.at[idx_v.at[pl.ds(0, CHUNK)]],
                                  rows[j], sem_g[j]).wait()

        def w_wait(j):
            pltpu.make_async_copy(rows[j], out.at[pl.ds(obase, CHUNK)],
                                  sem_w[j]).wait()

        for kc in range(min(nb, nchunks)):
            g_start(kc, kc)
        pend_w = [False] * nb
        for kc in range(nchunks):
            j = kc % nb
            g_wait(j)
            pltpu.async_copy(rows[j], out.at[pl.ds(obase + kc * CHUNK, CHUNK)],
                             sem_w[j])
            pend_w[j] = True
            if kc + nb < nchunks:
                w_wait(j)
                pend_w[j] = False
                g_start(kc + nb, j)
        for j in range(nb):
            if pend_w[j]:
                w_wait(j)

    return k


@functools.lru_cache(maxsize=None)
def _build_counts(ntot, etot):
    zr = ntot // NT
    nch = etot // NT // CHUNK
    assert zr % CHUNK == 0 and nch * NT * CHUNK == etot
    nv = zr // 16

    nb = _pick_nb(nch)

    @functools.partial(
        pl.kernel,
        out_type=jax.ShapeDtypeStruct((ntot,), _f32),
        mesh=_mesh(),
        scratch_types=(
            [pltpu.VMEM((CHUNK,), _f32),
             pltpu.VMEM((CHUNK,), _f32),
             pltpu.VMEM((nch, CHUNK), _i32),
             pltpu.VMEM((zr,), _f32),
             pltpu.VMEM((zr,), _f32),
             pltpu.VMEM_SHARED((ntot,), _f32)]
            + [pltpu.SemaphoreType.DMA for _ in range(nb)]
        ),
    )
    def k(dstl, zeros1, ones1, out, z_v, one_v, dst_v, cbuf, obuf, cnt, *sems):
        c = lax.axis_index("c")
        s = lax.axis_index("s")
        pltpu.sync_copy(zeros1, z_v)
        pltpu.sync_copy(ones1, one_v)
        base = s * zr
        for t in range(zr // CHUNK):
            pltpu.sync_copy(z_v, cnt.at[pl.ds(base + t * CHUNK, CHUNK)])
        pltpu.sync_copy(dstl.at[pl.ds(pl.multiple_of(s * nch, 8), nch)], dst_v)
        plsc.subcore_barrier()

        def s_start(kc, j):
            pltpu.async_copy(one_v, cnt.at[dst_v.at[kc]], sems[j], add=True)

        def s_wait(j):
            pltpu.make_async_copy(one_v, cnt.at[dst_v.at[0]], sems[j]).wait()

        def body(g, _):
            for j in range(nb):
                s_wait(j)
                s_start((g + 1) * nb + j, j)
            return _

        for j in range(nb):
            s_start(j, j)
        lax.fori_loop(0, nch // nb - 1, body, None)
        for j in range(nb):
            s_wait(j)
        plsc.subcore_barrier()
        pltpu.sync_copy(cnt.at[pl.ds(base, zr)], cbuf)
        for j in range(nv):
            v = cbuf[pl.ds(j * 16, 16)]
            obuf[pl.ds(j * 16, 16)] = 1.0 / jnp.maximum(v, 1.0)

        @pl.when(c == 0)
        def _():
            pltpu.sync_copy(obuf, out.at[pl.ds(base, zr)])

    return k


def _tc_mm(terms, bias, np_, relu=True, bn=512):
    k = len(terms)
    has_scale = tuple(sc is not None for _, _, sc in terms)
    grid = np_ // bn

    def body(*refs):
        i = 0
        acc = None
        for j in range(k):
            m_ref = refs[i]; w_ref = refs[i + 1]; i += 2
            p = jnp.dot(m_ref[...], w_ref[...], preferred_element_type=_f32, precision=lax.Precision.HIGHEST)
            if has_scale[j]:
                p = p * refs[i][...]
                i += 1
            acc = p if acc is None else acc + p
        acc = acc + refs[i][...]
        i += 1
        if relu:
            acc = jnp.maximum(acc, 0.0)
        out_ref = refs[i]
        out_ref[0] = acc[:, :128]
        out_ref[1] = acc[:, 128:]

    in_specs = []
    args = []
    for m, w, sc in terms:
        in_specs.append(pl.BlockSpec((bn, 128), lambda i: (i, 0)))
        args.append(m)
        in_specs.append(pl.BlockSpec((128, 256), lambda i: (0, 0)))
        args.append(w)
        if sc is not None:
            in_specs.append(pl.BlockSpec((bn, 1), lambda i: (i, 0)))
            args.append(sc)
    in_specs.append(pl.BlockSpec((1, 256), lambda i: (0, 0)))
    args.append(bias.reshape(1, 256))

    out = pl.pallas_call(
        body,
        grid=(grid,),
        in_specs=in_specs,
        out_specs=pl.BlockSpec((2, bn, 128), lambda i: (0, i, 0)),
        out_shape=jax.ShapeDtypeStruct((2, np_, 128), _f32),
    )(*args)
    return out.reshape(2 * np_, 128)


def _tc_head(gterms, b1, w2, b2):
    kk = len(gterms)

    def body(*refs):
        i = 0
        acc = None
        for j in range(kk):
            g = refs[i][...]; w = refs[i + 1][...]; sc = refs[i + 2][...]
            i += 3
            p = jnp.dot(g, w, preferred_element_type=_f32, precision=lax.Precision.HIGHEST) * sc
            acc = p if acc is None else acc + p
        h = jnp.maximum(acc + refs[i][...], 0.0)
        out = jnp.dot(h, refs[i + 1][...], preferred_element_type=_f32, precision=lax.Precision.HIGHEST) + refs[i + 2][...]
        refs[i + 3][...] = out

    args = []
    for g, w, sc in gterms:
        args += [g, w, sc]
    args += [b1.reshape(1, 256), w2, b2.reshape(1, 2)]
    return pl.pallas_call(
        body,
        out_shape=jax.ShapeDtypeStruct((128, 2), _f32),
    )(*args)


def _pad_rows(x, np_):
    return jnp.pad(x, ((0, np_ - x.shape[0]), (0, 0)))


def _colsplit(x, np_):
    n, d = x.shape
    cb = d // 128
    xp = _pad_rows(x, np_)
    return xp.reshape(np_, cb, 128).transpose(1, 0, 2).reshape(cb * np_, 128)


def _padi(idx, e_pad, fill):
    return jnp.pad(idx.astype(_i32), (0, e_pad - idx.shape[0]),
                   constant_values=fill)


def _split_w(w):
    return [w[i * 128:(i + 1) * 128] for i in range(w.shape[0] // 128)]


def kernel(x0, x1, x2, params, edge_index0, edge_index1, edge_index2,
           x0_batch, x1_batch, x2_batch, assign0, assign1):
    zeros128 = jnp.zeros((128, 128), _f32)
    zeros1 = jnp.zeros((128,), _f32)
    ones1 = jnp.ones((128,), _f32)

    X0 = _colsplit(x0, N0P)
    X1 = _colsplit(x1, N1P)
    X2 = _colsplit(x2, N2P)

    e0s = _padi(edge_index0[0], E0P, 0); e0d = _padi(edge_index0[1], E0P, N0)
    e1s = _padi(edge_index1[0], E1P, 0); e1d = _padi(edge_index1[1], E1P, N1)
    e2s = _padi(edge_index2[0], E2P, 0); e2d = _padi(edge_index2[1], E2P, N2)
    ar0 = _padi(jnp.arange(N0, dtype=_i32), A0E, 0)
    a0d = _padi(assign0, A0E, N1)
    ar1 = _padi(jnp.arange(N1, dtype=_i32), A1E, 0)
    a1d = _padi(assign1, A1E, N2)
    pb0s = _padi(jnp.arange(N0, dtype=_i32), B0E, 0)
    pb0d = _padi(x0_batch, B0E, NG)
    pb1s = _padi(jnp.arange(N1, dtype=_i32), B1E, 0)
    pb1d = _padi(x1_batch, B1E, NG)
    pb2s = _padi(jnp.arange(N2, dtype=_i32), B2E, 0)
    pb2d = _padi(x2_batch, B2E, NG)
    a0g = _padi(assign0, N0P, 0)
    a1g = _padi(assign1, N1P, 0)

    offs = [0, N0P, N0P + N1P, N0P + N1P + N2P, N0P + N1P + N2P + N1P,
            N0P + N1P + N2P + N1P + N2P]
    offs.append(offs[-1] + NGP)
    offs.append(offs[-1] + NGP)
    ntot = offs[-1] + NGP
    cat = jnp.concatenate([
        e0d, e1d + offs[1], e2d + offs[2], a0d + offs[3], a1d + offs[4],
        pb0d + offs[5], pb1d + offs[6], pb2d + offs[7]])
    etot_pad = ((cat.shape[0] + 16383) // 16384) * 16384
    cat = _padi(cat, etot_pad, ntot - 1)
    inv_all = _build_counts(ntot, etot_pad)(
        cat.reshape(-1, CHUNK), zeros1, ones1)
    inv_e0 = inv_all[offs[0]:offs[0] + N0P].reshape(-1, 1)
    inv_e1 = inv_all[offs[1]:offs[1] + N1P].reshape(-1, 1)
    inv_e2 = inv_all[offs[2]:offs[2] + N2P].reshape(-1, 1)
    inv_a0 = inv_all[offs[3]:offs[3] + N1P].reshape(-1, 1)
    inv_a1 = inv_all[offs[4]:offs[4] + N2P].reshape(-1, 1)
    inv_b0 = inv_all[offs[5]:offs[5] + 128].reshape(-1, 1)
    inv_b1 = inv_all[offs[6]:offs[6] + 128].reshape(-1, 1)
    inv_b2 = inv_all[offs[7]:offs[7] + 128].reshape(-1, 1)

    def blocks(h, np_, cb=2):
        return [h[i * np_:(i + 1) * np_] for i in range(cb)]

    def seg_idx(src, dst, cb, n_src_p, n_dst_p, edgesplit):
        if edgesplit:
            srcs, dst2 = src, dst
        else:
            srcs = jnp.concatenate([src + g * n_src_p for g in range(cb)])
            bps = cb // 2
            dst2 = (jnp.concatenate([dst + b * n_dst_p for b in range(bps)])
                    if bps > 1 else dst)
        return srcs, dst2.reshape(-1, CHUNK)

    def segsum(h, src, dst, cb, n_src_p, n_dst_p, e_pad, edgesplit=False,
               linear_src=False):
        srcs, dst2 = seg_idx(src, dst, cb, n_src_p, n_dst_p, edgesplit)
        if linear_src:
            srcs = srcs[:CHUNK]
        return _build_segsum(n_src_p, n_dst_p, cb, e_pad, edgesplit,
                             linear_src)(h, srcs, dst2, zeros128)

    def gcn(h, cb, np_, e_pad, srcl, dstl, inv, lp, edgesplit=False):
        s = segsum(h, srcl, dstl, cb, np_, np_, e_pad, edgesplit)
        sb = blocks(s, np_, 2 if edgesplit else cb)
        wn = _split_w(lp['Wn'])
        if edgesplit:
            wn = [wn[0], wn[0]]
        terms = [(m, w, inv) for m, w in zip(sb, wn)]
        terms += [(m, w, None) for m, w in zip(blocks(h, np_, cb), _split_w(lp['Ws']))]
        return _tc_mm(terms, lp['b'], np_)

    h0 = gcn(X0, 1, N0P, E0P, e0s, e0d, inv_e0, params['enc0_in'][0], edgesplit=True)
    h0 = gcn(h0, 2, N0P, E0P, e0s, e0d, inv_e0, params['enc0_in'][1])
    h1 = gcn(X1, 2, N1P, E1P, e1s, e1d, inv_e1, params['enc1_in'][0])
    h1 = gcn(h1, 2, N1P, E1P, e1s, e1d, inv_e1, params['enc1_in'][1])
    h2 = gcn(X2, 4, N2P, E2P, e2s, e2d, inv_e2, params['enc2_in'][0])
    h2 = gcn(h2, 2, N2P, E2P, e2s, e2d, inv_e2, params['enc2_in'][1])

    ip = params['inter']
    w0 = _split_w(ip['W0']); w1 = _split_w(ip['W1']); w2 = _split_w(ip['W2'])
    a0gc = jnp.concatenate([a0g, a0g + N1P])
    a1gc = jnp.concatenate([a1g, a1g + N2P])
    for _ in range(2):
        u01 = segsum(h0, ar0, a0d, 2, N0P, N1P, A0E, linear_src=True)
        u12 = segsum(h1, ar1, a1d, 2, N1P, N2P, A1E, linear_src=True)
        g01 = _build_gather(N1P, N0P)(h1, a0gc)
        g12 = _build_gather(N2P, N1P)(h2, a1gc)
        t0 = [(m, w, None) for m, w in zip(blocks(h0, N0P) + blocks(g01, N0P), w0)]
        t1 = ([(m, w, None) for m, w in zip(blocks(h1, N1P), w1[0:2])]
              + [(m, w, inv_a0) for m, w in zip(blocks(u01, N1P), w1[2:4])]
              + [(m, w, None) for m, w in zip(blocks(g12, N1P), w1[4:6])])
        t2 = ([(m, w, None) for m, w in zip(blocks(h2, N2P), w2[0:2])]
              + [(m, w, inv_a1) for m, w in zip(blocks(u12, N2P), w2[2:4])])
        h0 = _tc_mm(t0, ip['b0'], N0P)
        h1 = _tc_mm(t1, ip['b1'], N1P)
        h2 = _tc_mm(t2, ip['b2'], N2P)
        h0 = gcn(h0, 2, N0P, E0P, e0s, e0d, inv_e0, params['enc0'][0])
        h0 = gcn(h0, 2, N0P, E0P, e0s, e0d, inv_e0, params['enc0'][1])
        h1 = gcn(h1, 2, N1P, E1P, e1s, e1d, inv_e1, params['enc1'][0])
        h1 = gcn(h1, 2, N1P, E1P, e1s, e1d, inv_e1, params['enc1'][1])
        h2 = gcn(h2, 2, N2P, E2P, e2s, e2d, inv_e2, params['enc2'][0])
        h2 = gcn(h2, 2, N2P, E2P, e2s, e2d, inv_e2, params['enc2'][1])

    p0 = segsum(h0, pb0s, pb0d, 2, N0P, NGP, B0E, linear_src=True)
    p1 = segsum(h1, pb1s, pb1d, 2, N1P, NGP, B1E, linear_src=True)
    p2 = segsum(h2, pb2s, pb2d, 2, N2P, NGP, B2E, linear_src=True)

    hp = params['head']
    w1h = _split_w(hp['W1'])
    gterms = [
        (p0[0:128], w1h[0], inv_b0), (p0[NGP:NGP + 128], w1h[1], inv_b0),
        (p1[0:128], w1h[2], inv_b1), (p1[NGP:NGP + 128], w1h[3], inv_b1),
        (p2[0:128], w1h[4], inv_b2), (p2[NGP:NGP + 128], w1h[5], inv_b2),
    ]
    out = _tc_head(gterms, hp['b1'], hp['W2'], hp['b2'])
    return out[:NG]
